# baseline (speedup 1.0000x reference)
import jax
import jax.numpy as jnp
from jax import lax
from jax.experimental import pallas as pl
from jax.experimental.pallas import tpu as pltpu
from jax.experimental.pallas import tpu_sc as plsc

N = 100000
D = 128
CH = 96
NG = CH // 16
CB = D // 16
NCH = N // CH
TAIL = N - NCH * CH
NW = 32
TRIPS = -(-NCH // NW)


def _idx_copies(elem, arom, chg, hct, c, bufs, sem):
    ie, ia, ic, ih = bufs
    return [
        pltpu.make_async_copy(elem.at[pl.ds(c * CH, CH)], ie, sem),
        pltpu.make_async_copy(arom.at[pl.ds(c * CH, CH)], ia, sem),
        pltpu.make_async_copy(chg.at[pl.ds(c * CH, CH)], ic, sem),
        pltpu.make_async_copy(hct.at[pl.ds(c * CH, CH)], ih, sem),
    ]


def _assemble(bufs, rows, ngroups, tabs):
    iota = lax.iota(jnp.int32, 16)

    @plsc.parallel_loop(0, ngroups, unroll=2)
    def grp(g):
        base = g * 16
        for ibuf, tab, k in tabs:
            offs = ibuf[pl.ds(base, 16)] * D
            ss = [pl.multiple_of(offs[l], D) for l in range(16)]
            for l in range(16):
                row_src = tab.at[pl.ds(ss[l], D)]
                row_dst = rows.at[base + l, pl.ds(k * D, D)]
                for j in range(8):
                    row_dst[pl.ds(j * 16, 16)] = row_src[pl.ds(j * 16, 16)]


def _emb_body(elem, arom, chg, hct, We_h, Wa_h, Wc_h, Wh_h, out,
              iea, iaa, ica, iha, ieb, iab, icb, ihb, rows_a, rows_b,
              We, Wa, Wc, Wh,
              isa, isb, wsa, wsb):
    w = lax.axis_index("s") * 2 + lax.axis_index("c")
    bufs_a = (iea, iaa, ica, iha)
    bufs_b = (ieb, iab, icb, ihb)

    pltpu.sync_copy(We_h, We)
    pltpu.sync_copy(Wa_h, Wa)
    pltpu.sync_copy(Wc_h, Wc)
    pltpu.sync_copy(Wh_h, Wh)

    tabs_a = ((iea, We, 0), (iaa, Wa, 1), (ica, Wc, 2), (iha, Wh, 3))
    tabs_b = ((ieb, We, 0), (iab, Wa, 1), (icb, Wc, 2), (ihb, Wh, 3))

    def guard(t):
        return (t < TRIPS) & (t * NW + w < NCH)

    @pl.when(guard(0))
    def _():
        for cp in _idx_copies(elem, arom, chg, hct, 0 * NW + w, bufs_a, isa):
            cp.start()

    @pl.when(guard(1))
    def _():
        for cp in _idx_copies(elem, arom, chg, hct, 1 * NW + w, bufs_b, isb):
            cp.start()

    def half(t, c, bufs, tabs, rows, isem, wsem):
        @pl.when((t >= 2) & (c - 2 * NW < NCH))
        def _():
            pltpu.make_async_copy(out.at[pl.ds(0, CH)], rows, wsem).wait()

        @pl.when(guard(t))
        def _():
            for cp in _idx_copies(elem, arom, chg, hct, c, bufs, isem):
                cp.wait()
            _assemble(bufs, rows, NG, tabs)
            pltpu.make_async_copy(rows, out.at[pl.ds(c * CH, CH)],
                                  wsem).start()

        @pl.when(guard(t + 2))
        def _():
            for cp in _idx_copies(elem, arom, chg, hct, c + 2 * NW,
                                  bufs, isem):
                cp.start()

    def body2(j, carry):
        t0 = 2 * j
        t1 = t0 + 1
        half(t0, t0 * NW + w, bufs_a, tabs_a, rows_a, isa, wsa)
        half(t1, t1 * NW + w, bufs_b, tabs_b, rows_b, isb, wsb)
        return carry

    lax.fori_loop(0, (TRIPS + 1) // 2, body2, None)

    @pl.when(32 * NW + w < NCH)
    def _():
        pltpu.make_async_copy(out.at[pl.ds(0, CH)], rows_a, wsa).wait()

    @pl.when(w == NW - 1)
    def _():
        base = NCH * CH
        icps = [
            pltpu.make_async_copy(elem.at[pl.ds(base, TAIL)],
                                  iea.at[pl.ds(0, TAIL)], isa),
            pltpu.make_async_copy(arom.at[pl.ds(base, TAIL)],
                                  iaa.at[pl.ds(0, TAIL)], isa),
            pltpu.make_async_copy(chg.at[pl.ds(base, TAIL)],
                                  ica.at[pl.ds(0, TAIL)], isa),
            pltpu.make_async_copy(hct.at[pl.ds(base, TAIL)],
                                  iha.at[pl.ds(0, TAIL)], isa),
        ]
        for cp in icps:
            cp.start()
        for cp in icps:
            cp.wait()
        _assemble(bufs_a, rows_a, TAIL // 16, tabs_a)
        pltpu.sync_copy(rows_a.at[pl.ds(0, TAIL)], out.at[pl.ds(base, TAIL)])


def kernel(element, aromatic, charge, hcount,
           W_element, W_aromatic, W_charge, W_hcount):
    mesh = plsc.VectorSubcoreMesh(core_axis_name="c", subcore_axis_name="s")
    f = pl.kernel(
        _emb_body,
        mesh=mesh,
        out_type=jax.ShapeDtypeStruct((N, 4 * D), jnp.float32),
        scratch_types=[
            pltpu.VMEM((CH,), jnp.int32),
            pltpu.VMEM((CH,), jnp.int32),
            pltpu.VMEM((CH,), jnp.int32),
            pltpu.VMEM((CH,), jnp.int32),
            pltpu.VMEM((CH,), jnp.int32),
            pltpu.VMEM((CH,), jnp.int32),
            pltpu.VMEM((CH,), jnp.int32),
            pltpu.VMEM((CH,), jnp.int32),
            pltpu.VMEM((CH, 4 * D), jnp.float32),
            pltpu.VMEM((CH, 4 * D), jnp.float32),
            pltpu.VMEM((100 * D,), jnp.float32),
            pltpu.VMEM((2 * D,), jnp.float32),
            pltpu.VMEM((9 * D,), jnp.float32),
            pltpu.VMEM((9 * D,), jnp.float32),
            pltpu.SemaphoreType.DMA,
            pltpu.SemaphoreType.DMA,
            pltpu.SemaphoreType.DMA,
            pltpu.SemaphoreType.DMA,
        ],
    )
    return f(element, aromatic, charge, hcount,
             W_element.reshape(-1), W_aromatic.reshape(-1),
             W_charge.reshape(-1), W_hcount.reshape(-1))


# speedup vs baseline: 5.0171x; 5.0171x over previous
import jax
import jax.numpy as jnp
from jax import lax
from jax.experimental import pallas as pl
from jax.experimental.pallas import tpu as pltpu
from jax.experimental.pallas import tpu_sc as plsc

N = 100000
D = 128
CH = 96
NG = CH // 16
CB = D // 16
NCH = N // CH
TAIL = N - NCH * CH
NW = 32
TRIPS = -(-NCH // NW)


def _idx_copies(elem, arom, chg, hct, c, bufs, sem):
    ie, ia, ic, ih = bufs
    return [
        pltpu.make_async_copy(elem.at[pl.ds(c * CH, CH)], ie, sem),
        pltpu.make_async_copy(arom.at[pl.ds(c * CH, CH)], ia, sem),
        pltpu.make_async_copy(chg.at[pl.ds(c * CH, CH)], ic, sem),
        pltpu.make_async_copy(hct.at[pl.ds(c * CH, CH)], ih, sem),
    ]


def _assemble(bufs, rows, ngroups, tabs):
    iota = lax.iota(jnp.int32, 16)

    seq = [(t, j) for t in range(4) for j in range(8)]
    npf = 6

    def grp(g, carry):
        base = g * 16
        offv = [ibuf[pl.ds(base, 16)] * D for ibuf, _, _ in tabs]
        ss = [[pl.multiple_of(offv[t][l], D) for l in range(16)]
              for t in range(4)]
        for l in range(16):
            srcs = [tabs[t][1].at[pl.ds(ss[t][l], D)] for t in range(4)]
            dst = rows.at[base + l]
            xs = [None] * 32
            for i in range(npf):
                t, j = seq[i]
                xs[i] = srcs[t][pl.ds(j * 16, 16)]
            for i in range(32):
                if i + npf < 32:
                    t2, j2 = seq[i + npf]
                    xs[i + npf] = srcs[t2][pl.ds(j2 * 16, 16)]
                t, j = seq[i]
                dst[pl.ds(t * D + j * 16, 16)] = xs[i]
        return carry

    lax.fori_loop(0, ngroups, grp, None)


def _emb_body(elem, arom, chg, hct, We_h, Wa_h, Wc_h, Wh_h, out,
              iea, iaa, ica, iha, ieb, iab, icb, ihb, rows_a, rows_b,
              We, Wa, Wc, Wh,
              isa, isb, wsa, wsb):
    w = lax.axis_index("s") * 2 + lax.axis_index("c")
    bufs_a = (iea, iaa, ica, iha)
    bufs_b = (ieb, iab, icb, ihb)

    pltpu.sync_copy(We_h, We)
    pltpu.sync_copy(Wa_h, Wa)
    pltpu.sync_copy(Wc_h, Wc)
    pltpu.sync_copy(Wh_h, Wh)

    tabs_a = ((iea, We, 0), (iaa, Wa, 1), (ica, Wc, 2), (iha, Wh, 3))
    tabs_b = ((ieb, We, 0), (iab, Wa, 1), (icb, Wc, 2), (ihb, Wh, 3))

    def guard(t):
        return (t < TRIPS) & (t * NW + w < NCH)

    @pl.when(guard(0))
    def _():
        for cp in _idx_copies(elem, arom, chg, hct, 0 * NW + w, bufs_a, isa):
            cp.start()

    @pl.when(guard(1))
    def _():
        for cp in _idx_copies(elem, arom, chg, hct, 1 * NW + w, bufs_b, isb):
            cp.start()

    def half(t, c, bufs, tabs, rows, isem, wsem):
        @pl.when((t >= 2) & (c - 2 * NW < NCH))
        def _():
            pltpu.make_async_copy(out.at[pl.ds(0, CH)], rows, wsem).wait()

        @pl.when(guard(t))
        def _():
            for cp in _idx_copies(elem, arom, chg, hct, c, bufs, isem):
                cp.wait()
            _assemble(bufs, rows, NG, tabs)
            pltpu.make_async_copy(rows, out.at[pl.ds(c * CH, CH)],
                                  wsem).start()

        @pl.when(guard(t + 2))
        def _():
            for cp in _idx_copies(elem, arom, chg, hct, c + 2 * NW,
                                  bufs, isem):
                cp.start()

    def body2(j, carry):
        t0 = 2 * j
        t1 = t0 + 1
        half(t0, t0 * NW + w, bufs_a, tabs_a, rows_a, isa, wsa)
        half(t1, t1 * NW + w, bufs_b, tabs_b, rows_b, isb, wsb)
        return carry

    lax.fori_loop(0, (TRIPS + 1) // 2, body2, None)

    @pl.when(32 * NW + w < NCH)
    def _():
        pltpu.make_async_copy(out.at[pl.ds(0, CH)], rows_a, wsa).wait()

    @pl.when(w == NW - 1)
    def _():
        base = NCH * CH
        icps = [
            pltpu.make_async_copy(elem.at[pl.ds(base, TAIL)],
                                  iea.at[pl.ds(0, TAIL)], isa),
            pltpu.make_async_copy(arom.at[pl.ds(base, TAIL)],
                                  iaa.at[pl.ds(0, TAIL)], isa),
            pltpu.make_async_copy(chg.at[pl.ds(base, TAIL)],
                                  ica.at[pl.ds(0, TAIL)], isa),
            pltpu.make_async_copy(hct.at[pl.ds(base, TAIL)],
                                  iha.at[pl.ds(0, TAIL)], isa),
        ]
        for cp in icps:
            cp.start()
        for cp in icps:
            cp.wait()
        _assemble(bufs_a, rows_a, TAIL // 16, tabs_a)
        pltpu.sync_copy(rows_a.at[pl.ds(0, TAIL)], out.at[pl.ds(base, TAIL)])


def kernel(element, aromatic, charge, hcount,
           W_element, W_aromatic, W_charge, W_hcount):
    mesh = plsc.VectorSubcoreMesh(core_axis_name="c", subcore_axis_name="s")
    f = pl.kernel(
        _emb_body,
        mesh=mesh,
        out_type=jax.ShapeDtypeStruct((N, 4 * D), jnp.float32),
        scratch_types=[
            pltpu.VMEM((CH,), jnp.int32),
            pltpu.VMEM((CH,), jnp.int32),
            pltpu.VMEM((CH,), jnp.int32),
            pltpu.VMEM((CH,), jnp.int32),
            pltpu.VMEM((CH,), jnp.int32),
            pltpu.VMEM((CH,), jnp.int32),
            pltpu.VMEM((CH,), jnp.int32),
            pltpu.VMEM((CH,), jnp.int32),
            pltpu.VMEM((CH, 4 * D), jnp.float32),
            pltpu.VMEM((CH, 4 * D), jnp.float32),
            pltpu.VMEM((100 * D,), jnp.float32),
            pltpu.VMEM((2 * D,), jnp.float32),
            pltpu.VMEM((9 * D,), jnp.float32),
            pltpu.VMEM((9 * D,), jnp.float32),
            pltpu.SemaphoreType.DMA,
            pltpu.SemaphoreType.DMA,
            pltpu.SemaphoreType.DMA,
            pltpu.SemaphoreType.DMA,
        ],
    )
    return f(element, aromatic, charge, hcount,
             W_element.reshape(-1), W_aromatic.reshape(-1),
             W_charge.reshape(-1), W_hcount.reshape(-1))


# speedup vs baseline: 5.5688x; 1.1100x over previous
import jax
import jax.numpy as jnp
from jax import lax
from jax.experimental import pallas as pl
from jax.experimental.pallas import tpu as pltpu
from jax.experimental.pallas import tpu_sc as plsc

N = 100000
D = 128
CH = 96
NG = CH // 16
CB = D // 16
NCH = N // CH
TAIL = N - NCH * CH
NW = 32
TRIPS = -(-NCH // NW)


def _idx_copies(elem, arom, chg, hct, c, bufs, sem):
    ie, ia, ic, ih = bufs
    return [
        pltpu.make_async_copy(elem.at[pl.ds(c * CH, CH)], ie, sem),
        pltpu.make_async_copy(arom.at[pl.ds(c * CH, CH)], ia, sem),
        pltpu.make_async_copy(chg.at[pl.ds(c * CH, CH)], ic, sem),
        pltpu.make_async_copy(hct.at[pl.ds(c * CH, CH)], ih, sem),
    ]


def _assemble(bufs, rows, ngroups, tabs):
    iota = lax.iota(jnp.int32, 16)

    seq = [(l, t, j) for l in range(16) for t in range(4) for j in range(8)]
    npf = 6

    def grp(g, carry):
        base = g * 16
        offv = [ibuf[pl.ds(base, 16)] * D for ibuf, _, _ in tabs]
        ss = [[pl.multiple_of(offv[t][l], D) for l in range(16)]
              for t in range(4)]
        srcs = [[tabs[t][1].at[pl.ds(ss[t][l], D)] for t in range(4)]
                for l in range(16)]
        dsts = [rows.at[base + l] for l in range(16)]
        nseq = len(seq)
        xs = [None] * nseq
        for i in range(npf):
            l, t, j = seq[i]
            xs[i] = srcs[l][t][pl.ds(j * 16, 16)]
        for i in range(nseq):
            if i + npf < nseq:
                l2, t2, j2 = seq[i + npf]
                xs[i + npf] = srcs[l2][t2][pl.ds(j2 * 16, 16)]
            l, t, j = seq[i]
            dsts[l][pl.ds(t * D + j * 16, 16)] = xs[i]
        return carry

    lax.fori_loop(0, ngroups, grp, None)


def _emb_body(elem, arom, chg, hct, We_h, Wa_h, Wc_h, Wh_h, out,
              iea, iaa, ica, iha, ieb, iab, icb, ihb, rows_a, rows_b,
              We, Wa, Wc, Wh,
              isa, isb, wsa, wsb):
    w = lax.axis_index("s") * 2 + lax.axis_index("c")
    bufs_a = (iea, iaa, ica, iha)
    bufs_b = (ieb, iab, icb, ihb)

    pltpu.sync_copy(We_h, We)
    pltpu.sync_copy(Wa_h, Wa)
    pltpu.sync_copy(Wc_h, Wc)
    pltpu.sync_copy(Wh_h, Wh)

    tabs_a = ((iea, We, 0), (iaa, Wa, 1), (ica, Wc, 2), (iha, Wh, 3))
    tabs_b = ((ieb, We, 0), (iab, Wa, 1), (icb, Wc, 2), (ihb, Wh, 3))

    def guard(t):
        return (t < TRIPS) & (t * NW + w < NCH)

    @pl.when(guard(0))
    def _():
        for cp in _idx_copies(elem, arom, chg, hct, 0 * NW + w, bufs_a, isa):
            cp.start()

    @pl.when(guard(1))
    def _():
        for cp in _idx_copies(elem, arom, chg, hct, 1 * NW + w, bufs_b, isb):
            cp.start()

    def half(t, c, bufs, tabs, rows, isem, wsem):
        @pl.when((t >= 2) & (c - 2 * NW < NCH))
        def _():
            pltpu.make_async_copy(out.at[pl.ds(0, CH)], rows, wsem).wait()

        @pl.when(guard(t))
        def _():
            for cp in _idx_copies(elem, arom, chg, hct, c, bufs, isem):
                cp.wait()
            _assemble(bufs, rows, NG, tabs)
            pltpu.make_async_copy(rows, out.at[pl.ds(c * CH, CH)],
                                  wsem).start()

        @pl.when(guard(t + 2))
        def _():
            for cp in _idx_copies(elem, arom, chg, hct, c + 2 * NW,
                                  bufs, isem):
                cp.start()

    def body2(j, carry):
        t0 = 2 * j
        t1 = t0 + 1
        half(t0, t0 * NW + w, bufs_a, tabs_a, rows_a, isa, wsa)
        half(t1, t1 * NW + w, bufs_b, tabs_b, rows_b, isb, wsb)
        return carry

    lax.fori_loop(0, (TRIPS + 1) // 2, body2, None)

    @pl.when(32 * NW + w < NCH)
    def _():
        pltpu.make_async_copy(out.at[pl.ds(0, CH)], rows_a, wsa).wait()

    @pl.when(w == NW - 1)
    def _():
        base = NCH * CH
        icps = [
            pltpu.make_async_copy(elem.at[pl.ds(base, TAIL)],
                                  iea.at[pl.ds(0, TAIL)], isa),
            pltpu.make_async_copy(arom.at[pl.ds(base, TAIL)],
                                  iaa.at[pl.ds(0, TAIL)], isa),
            pltpu.make_async_copy(chg.at[pl.ds(base, TAIL)],
                                  ica.at[pl.ds(0, TAIL)], isa),
            pltpu.make_async_copy(hct.at[pl.ds(base, TAIL)],
                                  iha.at[pl.ds(0, TAIL)], isa),
        ]
        for cp in icps:
            cp.start()
        for cp in icps:
            cp.wait()
        _assemble(bufs_a, rows_a, TAIL // 16, tabs_a)
        pltpu.sync_copy(rows_a.at[pl.ds(0, TAIL)], out.at[pl.ds(base, TAIL)])


def kernel(element, aromatic, charge, hcount,
           W_element, W_aromatic, W_charge, W_hcount):
    mesh = plsc.VectorSubcoreMesh(core_axis_name="c", subcore_axis_name="s")
    f = pl.kernel(
        _emb_body,
        mesh=mesh,
        out_type=jax.ShapeDtypeStruct((N, 4 * D), jnp.float32),
        scratch_types=[
            pltpu.VMEM((CH,), jnp.int32),
            pltpu.VMEM((CH,), jnp.int32),
            pltpu.VMEM((CH,), jnp.int32),
            pltpu.VMEM((CH,), jnp.int32),
            pltpu.VMEM((CH,), jnp.int32),
            pltpu.VMEM((CH,), jnp.int32),
            pltpu.VMEM((CH,), jnp.int32),
            pltpu.VMEM((CH,), jnp.int32),
            pltpu.VMEM((CH, 4 * D), jnp.float32),
            pltpu.VMEM((CH, 4 * D), jnp.float32),
            pltpu.VMEM((100 * D,), jnp.float32),
            pltpu.VMEM((2 * D,), jnp.float32),
            pltpu.VMEM((9 * D,), jnp.float32),
            pltpu.VMEM((9 * D,), jnp.float32),
            pltpu.SemaphoreType.DMA,
            pltpu.SemaphoreType.DMA,
            pltpu.SemaphoreType.DMA,
            pltpu.SemaphoreType.DMA,
        ],
    )
    return f(element, aromatic, charge, hcount,
             W_element.reshape(-1), W_aromatic.reshape(-1),
             W_charge.reshape(-1), W_hcount.reshape(-1))


# speedup vs baseline: 5.5853x; 1.0030x over previous
import jax
import jax.numpy as jnp
from jax import lax
from jax.experimental import pallas as pl
from jax.experimental.pallas import tpu as pltpu
from jax.experimental.pallas import tpu_sc as plsc

N = 100000
D = 128
CH = 96
NG = CH // 16
CB = D // 16
NCH = N // CH
TAIL = N - NCH * CH
NW = 32
TRIPS = -(-NCH // NW)


def _idx_copies(elem, arom, chg, hct, c, bufs, sem):
    ie, ia, ic, ih = bufs
    return [
        pltpu.make_async_copy(elem.at[pl.ds(c * CH, CH)], ie, sem),
        pltpu.make_async_copy(arom.at[pl.ds(c * CH, CH)], ia, sem),
        pltpu.make_async_copy(chg.at[pl.ds(c * CH, CH)], ic, sem),
        pltpu.make_async_copy(hct.at[pl.ds(c * CH, CH)], ih, sem),
    ]


def _assemble(bufs, rows, ngroups, tabs):
    iota = lax.iota(jnp.int32, 16)

    seq = [(l, t, j) for l in range(16) for t in range(4) for j in range(8)]
    npf = 10

    def grp(g, carry):
        base = g * 16
        offv = [ibuf[pl.ds(base, 16)] * D for ibuf, _, _ in tabs]
        ss = [[pl.multiple_of(offv[t][l], D) for l in range(16)]
              for t in range(4)]
        srcs = [[tabs[t][1].at[pl.ds(ss[t][l], D)] for t in range(4)]
                for l in range(16)]
        dsts = [rows.at[base + l] for l in range(16)]
        nseq = len(seq)
        xs = [None] * nseq
        for i in range(npf):
            l, t, j = seq[i]
            xs[i] = srcs[l][t][pl.ds(j * 16, 16)]
        for i in range(nseq):
            if i + npf < nseq:
                l2, t2, j2 = seq[i + npf]
                xs[i + npf] = srcs[l2][t2][pl.ds(j2 * 16, 16)]
            l, t, j = seq[i]
            dsts[l][pl.ds(t * D + j * 16, 16)] = xs[i]
        return carry

    lax.fori_loop(0, ngroups, grp, None)


def _emb_body(elem, arom, chg, hct, We_h, Wa_h, Wc_h, Wh_h, out,
              iea, iaa, ica, iha, ieb, iab, icb, ihb, rows_a, rows_b,
              We, Wa, Wc, Wh,
              isa, isb, wsa, wsb):
    w = lax.axis_index("s") * 2 + lax.axis_index("c")
    bufs_a = (iea, iaa, ica, iha)
    bufs_b = (ieb, iab, icb, ihb)

    pltpu.sync_copy(We_h, We)
    pltpu.sync_copy(Wa_h, Wa)
    pltpu.sync_copy(Wc_h, Wc)
    pltpu.sync_copy(Wh_h, Wh)

    tabs_a = ((iea, We, 0), (iaa, Wa, 1), (ica, Wc, 2), (iha, Wh, 3))
    tabs_b = ((ieb, We, 0), (iab, Wa, 1), (icb, Wc, 2), (ihb, Wh, 3))

    def guard(t):
        return (t < TRIPS) & (t * NW + w < NCH)

    @pl.when(guard(0))
    def _():
        for cp in _idx_copies(elem, arom, chg, hct, 0 * NW + w, bufs_a, isa):
            cp.start()

    @pl.when(guard(1))
    def _():
        for cp in _idx_copies(elem, arom, chg, hct, 1 * NW + w, bufs_b, isb):
            cp.start()

    def half(t, c, bufs, tabs, rows, isem, wsem):
        @pl.when((t >= 2) & (c - 2 * NW < NCH))
        def _():
            pltpu.make_async_copy(out.at[pl.ds(0, CH)], rows, wsem).wait()

        @pl.when(guard(t))
        def _():
            for cp in _idx_copies(elem, arom, chg, hct, c, bufs, isem):
                cp.wait()
            _assemble(bufs, rows, NG, tabs)
            pltpu.make_async_copy(rows, out.at[pl.ds(c * CH, CH)],
                                  wsem).start()

        @pl.when(guard(t + 2))
        def _():
            for cp in _idx_copies(elem, arom, chg, hct, c + 2 * NW,
                                  bufs, isem):
                cp.start()

    def body2(j, carry):
        t0 = 2 * j
        t1 = t0 + 1
        half(t0, t0 * NW + w, bufs_a, tabs_a, rows_a, isa, wsa)
        half(t1, t1 * NW + w, bufs_b, tabs_b, rows_b, isb, wsb)
        return carry

    lax.fori_loop(0, (TRIPS + 1) // 2, body2, None)

    @pl.when(32 * NW + w < NCH)
    def _():
        pltpu.make_async_copy(out.at[pl.ds(0, CH)], rows_a, wsa).wait()

    @pl.when(w == NW - 1)
    def _():
        base = NCH * CH
        icps = [
            pltpu.make_async_copy(elem.at[pl.ds(base, TAIL)],
                                  iea.at[pl.ds(0, TAIL)], isa),
            pltpu.make_async_copy(arom.at[pl.ds(base, TAIL)],
                                  iaa.at[pl.ds(0, TAIL)], isa),
            pltpu.make_async_copy(chg.at[pl.ds(base, TAIL)],
                                  ica.at[pl.ds(0, TAIL)], isa),
            pltpu.make_async_copy(hct.at[pl.ds(base, TAIL)],
                                  iha.at[pl.ds(0, TAIL)], isa),
        ]
        for cp in icps:
            cp.start()
        for cp in icps:
            cp.wait()
        _assemble(bufs_a, rows_a, TAIL // 16, tabs_a)
        pltpu.sync_copy(rows_a.at[pl.ds(0, TAIL)], out.at[pl.ds(base, TAIL)])


def kernel(element, aromatic, charge, hcount,
           W_element, W_aromatic, W_charge, W_hcount):
    mesh = plsc.VectorSubcoreMesh(core_axis_name="c", subcore_axis_name="s")
    f = pl.kernel(
        _emb_body,
        mesh=mesh,
        out_type=jax.ShapeDtypeStruct((N, 4 * D), jnp.float32),
        scratch_types=[
            pltpu.VMEM((CH,), jnp.int32),
            pltpu.VMEM((CH,), jnp.int32),
            pltpu.VMEM((CH,), jnp.int32),
            pltpu.VMEM((CH,), jnp.int32),
            pltpu.VMEM((CH,), jnp.int32),
            pltpu.VMEM((CH,), jnp.int32),
            pltpu.VMEM((CH,), jnp.int32),
            pltpu.VMEM((CH,), jnp.int32),
            pltpu.VMEM((CH, 4 * D), jnp.float32),
            pltpu.VMEM((CH, 4 * D), jnp.float32),
            pltpu.VMEM((100 * D,), jnp.float32),
            pltpu.VMEM((2 * D,), jnp.float32),
            pltpu.VMEM((9 * D,), jnp.float32),
            pltpu.VMEM((9 * D,), jnp.float32),
            pltpu.SemaphoreType.DMA,
            pltpu.SemaphoreType.DMA,
            pltpu.SemaphoreType.DMA,
            pltpu.SemaphoreType.DMA,
        ],
    )
    return f(element, aromatic, charge, hcount,
             W_element.reshape(-1), W_aromatic.reshape(-1),
             W_charge.reshape(-1), W_hcount.reshape(-1))


# speedup vs baseline: 5.5932x; 1.0014x over previous
import jax
import jax.numpy as jnp
from jax import lax
from jax.experimental import pallas as pl
from jax.experimental.pallas import tpu as pltpu
from jax.experimental.pallas import tpu_sc as plsc

N = 100000
D = 128
CH = 96
NG = CH // 16
NCH = N // CH
TAIL = N - NCH * CH
NW = 32
TRIPS = -(-NCH // NW)


def _idx_copies(elem, arom, chg, hct, c, bufs, sem):
    ie, ia, ic, ih = bufs
    return [
        pltpu.make_async_copy(elem.at[pl.ds(c * CH, CH)], ie, sem),
        pltpu.make_async_copy(arom.at[pl.ds(c * CH, CH)], ia, sem),
        pltpu.make_async_copy(chg.at[pl.ds(c * CH, CH)], ic, sem),
        pltpu.make_async_copy(hct.at[pl.ds(c * CH, CH)], ih, sem),
    ]


def _assemble(bufs, rows, ngroups, tabs):
    seq = [(l, t, j) for l in range(16) for t in range(4) for j in range(8)]
    npf = 10

    def grp(g, carry):
        base = g * 16
        offv = [ibuf[pl.ds(base, 16)] * D for ibuf, _, _ in tabs]
        ss = [[pl.multiple_of(offv[t][l], D) for l in range(16)]
              for t in range(4)]
        srcs = [[tabs[t][1].at[pl.ds(ss[t][l], D)] for t in range(4)]
                for l in range(16)]
        dsts = [rows.at[base + l] for l in range(16)]
        nseq = len(seq)
        xs = [None] * nseq
        for i in range(npf):
            l, t, j = seq[i]
            xs[i] = srcs[l][t][pl.ds(j * 16, 16)]
        for i in range(nseq):
            if i + npf < nseq:
                l2, t2, j2 = seq[i + npf]
                xs[i + npf] = srcs[l2][t2][pl.ds(j2 * 16, 16)]
            l, t, j = seq[i]
            dsts[l][pl.ds(t * D + j * 16, 16)] = xs[i]
        return carry

    lax.fori_loop(0, ngroups, grp, None)


def _emb_body(elem, arom, chg, hct, We_h, Wa_h, Wc_h, Wh_h, out,
              iea, iaa, ica, iha, ieb, iab, icb, ihb, rows_a, rows_b,
              We, Wa, Wc, Wh,
              isa, isb, wsa, wsb):
    w = lax.axis_index("s") * 2 + lax.axis_index("c")
    bufs_a = (iea, iaa, ica, iha)
    bufs_b = (ieb, iab, icb, ihb)

    pltpu.sync_copy(We_h, We)
    pltpu.sync_copy(Wa_h, Wa)
    pltpu.sync_copy(Wc_h, Wc)
    pltpu.sync_copy(Wh_h, Wh)

    tabs_a = ((iea, We, 0), (iaa, Wa, 1), (ica, Wc, 2), (iha, Wh, 3))
    tabs_b = ((ieb, We, 0), (iab, Wa, 1), (icb, Wc, 2), (ihb, Wh, 3))

    def guard(t):
        return (t < TRIPS) & (t * NW + w < NCH)

    @pl.when(guard(0))
    def _():
        for cp in _idx_copies(elem, arom, chg, hct, 0 * NW + w, bufs_a, isa):
            cp.start()

    @pl.when(guard(1))
    def _():
        for cp in _idx_copies(elem, arom, chg, hct, 1 * NW + w, bufs_b, isb):
            cp.start()

    def half(t, c, bufs, tabs, rows, isem, wsem):
        @pl.when((t >= 2) & (c - 2 * NW < NCH))
        def _():
            pltpu.make_async_copy(out.at[pl.ds(0, CH)], rows, wsem).wait()

        @pl.when(guard(t))
        def _():
            for cp in _idx_copies(elem, arom, chg, hct, c, bufs, isem):
                cp.wait()
            _assemble(bufs, rows, NG, tabs)
            pltpu.make_async_copy(rows, out.at[pl.ds(c * CH, CH)],
                                  wsem).start()

        @pl.when(guard(t + 2))
        def _():
            for cp in _idx_copies(elem, arom, chg, hct, c + 2 * NW,
                                  bufs, isem):
                cp.start()

    def body2(j, carry):
        t0 = 2 * j
        t1 = t0 + 1
        half(t0, t0 * NW + w, bufs_a, tabs_a, rows_a, isa, wsa)
        half(t1, t1 * NW + w, bufs_b, tabs_b, rows_b, isb, wsb)
        return carry

    lax.fori_loop(0, (TRIPS + 1) // 2, body2, None)

    @pl.when(32 * NW + w < NCH)
    def _():
        pltpu.make_async_copy(out.at[pl.ds(0, CH)], rows_a, wsa).wait()

    @pl.when(w == NW - 1)
    def _():
        base = NCH * CH
        icps = [
            pltpu.make_async_copy(elem.at[pl.ds(base, TAIL)],
                                  iea.at[pl.ds(0, TAIL)], isa),
            pltpu.make_async_copy(arom.at[pl.ds(base, TAIL)],
                                  iaa.at[pl.ds(0, TAIL)], isa),
            pltpu.make_async_copy(chg.at[pl.ds(base, TAIL)],
                                  ica.at[pl.ds(0, TAIL)], isa),
            pltpu.make_async_copy(hct.at[pl.ds(base, TAIL)],
                                  iha.at[pl.ds(0, TAIL)], isa),
        ]
        for cp in icps:
            cp.start()
        for cp in icps:
            cp.wait()
        _assemble(bufs_a, rows_a, TAIL // 16, tabs_a)
        pltpu.sync_copy(rows_a.at[pl.ds(0, TAIL)], out.at[pl.ds(base, TAIL)])


def kernel(element, aromatic, charge, hcount,
           W_element, W_aromatic, W_charge, W_hcount):
    mesh = plsc.VectorSubcoreMesh(core_axis_name="c", subcore_axis_name="s")
    f = pl.kernel(
        _emb_body,
        mesh=mesh,
        out_type=jax.ShapeDtypeStruct((N, 4 * D), jnp.float32),
        scratch_types=[
            pltpu.VMEM((CH,), jnp.int32),
            pltpu.VMEM((CH,), jnp.int32),
            pltpu.VMEM((CH,), jnp.int32),
            pltpu.VMEM((CH,), jnp.int32),
            pltpu.VMEM((CH,), jnp.int32),
            pltpu.VMEM((CH,), jnp.int32),
            pltpu.VMEM((CH,), jnp.int32),
            pltpu.VMEM((CH,), jnp.int32),
            pltpu.VMEM((CH, 4 * D), jnp.float32),
            pltpu.VMEM((CH, 4 * D), jnp.float32),
            pltpu.VMEM((100 * D,), jnp.float32),
            pltpu.VMEM((2 * D,), jnp.float32),
            pltpu.VMEM((9 * D,), jnp.float32),
            pltpu.VMEM((9 * D,), jnp.float32),
            pltpu.SemaphoreType.DMA,
            pltpu.SemaphoreType.DMA,
            pltpu.SemaphoreType.DMA,
            pltpu.SemaphoreType.DMA,
        ],
    )
    return f(element, aromatic, charge, hcount,
             W_element.reshape(-1), W_aromatic.reshape(-1),
             W_charge.reshape(-1), W_hcount.reshape(-1))
